# Initial kernel scaffold; baseline (speedup 1.0000x reference)
import jax
import jax.numpy as jnp
from jax.experimental import pallas as pl


def kernel(token_window, bank):
    raise NotImplementedError("write your pallas kernel here")


# speedup vs baseline: 4.2271x; 4.2271x over previous
import functools

import jax
import jax.numpy as jnp
from jax import lax
from jax.experimental import pallas as pl
from jax.experimental.pallas import tpu as pltpu
from jax.experimental.pallas import tpu_sc as plsc

N_GRAM = 4
D_STATE = 16
N_SLOTS = 500000
N_BLADES = 8

NC = 2
NS = 16
L = 16
NW = NC * NS

FNV_INIT = -2128831035
FNV_PRIME = 16777619
HIGH_MOD = 483648

N_TOK = 1024 * 200
PER_W = N_TOK // NW
CHUNK = 256
N_CHUNKS = PER_W // CHUNK
ROWS_CHUNK = CHUNK * N_BLADES


def _tile_body(tok_hbm, bank_hbm, out_hbm, tok_v, idx_v, rows_v, sem):
    wid = (lax.axis_index("s").astype(jnp.int32) * jnp.int32(NC)
           + lax.axis_index("c").astype(jnp.int32))
    base = wid * jnp.int32(PER_W)
    for j in range(N_GRAM):
        pltpu.sync_copy(tok_hbm.at[jnp.int32(j), pl.ds(base, PER_W)],
                        tok_v.at[jnp.int32(j)])
    lane8 = lax.iota(jnp.int32, L) * N_BLADES

    def chunk_body(c, carry):
        c = c.astype(jnp.int32)

        def vec_body(i, carry2):
            i = i.astype(jnp.int32)
            off = c * jnp.int32(CHUNK) + i * jnp.int32(L)
            h = jnp.full((L,), FNV_INIT, dtype=jnp.int32)
            for j in range(N_GRAM):
                h = (h ^ tok_v[jnp.int32(j), pl.ds(off, L)]) * jnp.int32(FNV_PRIME)
            m = lax.rem(h & jnp.int32(0x7FFFFFFF), jnp.int32(N_SLOTS))
            fix = jnp.where(h < 0, jnp.int32(HIGH_MOD), jnp.int32(0))
            addr = lax.rem(m + fix, jnp.int32(N_SLOTS))
            pos = i * jnp.int32(L * N_BLADES) + lane8
            for k in range(N_BLADES):
                plsc.store_scatter(idx_v, [pos + jnp.int32(k)],
                                   addr + jnp.int32(k * N_SLOTS))
            return carry2

        lax.fori_loop(jnp.int32(0), jnp.int32(CHUNK // L), vec_body, 0)
        pltpu.async_copy(bank_hbm.at[idx_v], rows_v, sem).wait()
        out_row = wid * jnp.int32(N_CHUNKS * ROWS_CHUNK) + c * jnp.int32(ROWS_CHUNK)
        pltpu.sync_copy(rows_v, out_hbm.at[pl.ds(out_row, ROWS_CHUNK)])
        return carry

    lax.fori_loop(jnp.int32(0), jnp.int32(N_CHUNKS), chunk_body, 0)


_sc_gather = functools.partial(
    pl.kernel,
    mesh=plsc.VectorSubcoreMesh(core_axis_name="c", subcore_axis_name="s"),
    compiler_params=pltpu.CompilerParams(needs_layout_passes=False,
                                         use_tc_tiling_on_sc=False),
    out_type=jax.ShapeDtypeStruct((N_TOK * N_BLADES, D_STATE), jnp.float32),
    scratch_types=[
        pltpu.VMEM((N_GRAM, PER_W), jnp.int32),
        pltpu.VMEM((ROWS_CHUNK,), jnp.int32),
        pltpu.VMEM((ROWS_CHUNK, D_STATE), jnp.float32),
        pltpu.SemaphoreType.DMA,
    ],
)(_tile_body)


def kernel(token_window, bank):
    B, S = token_window.shape[:2]
    tok = token_window.astype(jnp.int32).reshape(-1, N_GRAM).T
    bank2d = bank.reshape(N_BLADES * N_SLOTS, D_STATE)
    out = _sc_gather(tok, bank2d)
    return out.reshape(B, S, N_BLADES, D_STATE)


# speedup vs baseline: 27.4015x; 6.4823x over previous
import functools

import jax
import jax.numpy as jnp
from jax import lax
from jax.experimental import pallas as pl
from jax.experimental.pallas import tpu as pltpu
from jax.experimental.pallas import tpu_sc as plsc

N_GRAM = 4
D_STATE = 16
N_SLOTS = 500000
N_BLADES = 8

NC = 2
NS = 16
L = 16
NW = NC * NS

FNV_INIT = -2128831035
FNV_PRIME = 16777619
HIGH_MOD = 483648

B_DIM = 1024
S_DIM = 200
N_TOK = B_DIM * S_DIM
PER_W = N_TOK // NW
B_PER_W = PER_W // S_DIM
CHUNK = 2 * S_DIM
N_CHUNKS = PER_W // CHUNK


def _hash_chunk(tok_v, idx_ref, c):
    def vec_body(i, carry):
        i = i.astype(jnp.int32)
        off = jnp.int32(c * CHUNK) + i * jnp.int32(L)
        h = jnp.full((L,), FNV_INIT, dtype=jnp.int32)
        for j in range(N_GRAM):
            h = (h ^ tok_v[jnp.int32(j), pl.ds(off, L)]) * jnp.int32(FNV_PRIME)
        m = lax.rem(h & jnp.int32(0x7FFFFFFF), jnp.int32(N_SLOTS))
        fix = jnp.where(h < 0, jnp.int32(HIGH_MOD), jnp.int32(0))
        idx_ref[pl.ds(i * jnp.int32(L), L)] = lax.rem(m + fix,
                                                      jnp.int32(N_SLOTS))
        return carry

    lax.fori_loop(jnp.int32(0), jnp.int32(CHUNK // L), vec_body, 0)


def _tile_body(tok_hbm, bank_hbm, out_hbm, tok_v, idx0, idx1, rows0, rows1,
               sem0, sem1):
    idx = (idx0, idx1)
    rows = (rows0, rows1)
    sems = (sem0, sem1)
    wid = (lax.axis_index("s").astype(jnp.int32) * jnp.int32(NC)
           + lax.axis_index("c").astype(jnp.int32))
    base = pl.multiple_of(wid * jnp.int32(PER_W), 128)
    for j in range(N_GRAM):
        pltpu.sync_copy(tok_hbm.at[jnp.int32(j), pl.ds(base, PER_W)],
                        tok_v.at[jnp.int32(j)])
    b_base = wid * jnp.int32(B_PER_W)

    def start_gather(r):
        seg = idx[(r // 2) % 2].at[pl.ds((r % 2) * S_DIM, S_DIM)]
        return pltpu.async_copy(bank_hbm.at[seg], rows[r % 2], sems[r % 2])

    _hash_chunk(tok_v, idx[0], 0)
    pending = start_gather(0)
    for r in range(B_PER_W):
        nxt = None
        if r + 1 < B_PER_W:
            if (r + 1) % 2 == 0:
                _hash_chunk(tok_v, idx[((r + 1) // 2) % 2], (r + 1) // 2)
            nxt = start_gather(r + 1)
        pending.wait()
        pltpu.sync_copy(rows[r % 2], out_hbm.at[b_base + jnp.int32(r)])
        pending = nxt


_sc_gather = functools.partial(
    pl.kernel,
    mesh=plsc.VectorSubcoreMesh(core_axis_name="c", subcore_axis_name="s"),
    compiler_params=pltpu.CompilerParams(needs_layout_passes=False,
                                         use_tc_tiling_on_sc=True),
    out_type=jax.ShapeDtypeStruct((B_DIM, S_DIM, N_BLADES * D_STATE),
                                  jnp.float32),
    scratch_types=[
        pltpu.VMEM((N_GRAM, PER_W), jnp.int32),
        pltpu.VMEM((CHUNK,), jnp.int32),
        pltpu.VMEM((CHUNK,), jnp.int32),
        pltpu.VMEM((S_DIM, N_BLADES * D_STATE), jnp.float32),
        pltpu.VMEM((S_DIM, N_BLADES * D_STATE), jnp.float32),
        pltpu.SemaphoreType.DMA,
        pltpu.SemaphoreType.DMA,
    ],
)(_tile_body)


def kernel(token_window, bank):
    tok = token_window.astype(jnp.int32).reshape(-1, N_GRAM).T
    bank_t = jnp.transpose(bank, (1, 0, 2)).reshape(N_SLOTS,
                                                    N_BLADES * D_STATE)
    out = _sc_gather(tok, bank_t)
    return out.reshape(B_DIM, S_DIM, N_BLADES, D_STATE)


# speedup vs baseline: 27.5184x; 1.0043x over previous
import functools

import jax
import jax.numpy as jnp
from jax import lax
from jax.experimental import pallas as pl
from jax.experimental.pallas import tpu as pltpu
from jax.experimental.pallas import tpu_sc as plsc

N_GRAM = 4
D_STATE = 16
N_SLOTS = 500000
N_BLADES = 8

NC = 2
NS = 16
L = 16
NW = NC * NS

FNV_INIT = -2128831035
FNV_PRIME = 16777619
HIGH_MOD = 483648

B_DIM = 1024
S_DIM = 200
N_TOK = B_DIM * S_DIM
PER_W = N_TOK // NW
B_PER_W = PER_W // S_DIM
CHUNK = 2 * S_DIM
N_CHUNKS = PER_W // CHUNK


def _hash_chunk(tok_v, idx_ref, c):
    def vec_body(i, carry):
        i = i.astype(jnp.int32)
        off = jnp.int32(c * CHUNK) + i * jnp.int32(L)
        h = jnp.full((L,), FNV_INIT, dtype=jnp.int32)
        for j in range(N_GRAM):
            h = (h ^ tok_v[jnp.int32(j), pl.ds(off, L)]) * jnp.int32(FNV_PRIME)
        m = lax.rem(h & jnp.int32(0x7FFFFFFF), jnp.int32(N_SLOTS))
        fix = jnp.where(h < 0, jnp.int32(HIGH_MOD), jnp.int32(0))
        idx_ref[pl.ds(i * jnp.int32(L), L)] = lax.rem(m + fix,
                                                      jnp.int32(N_SLOTS))
        return carry

    lax.fori_loop(jnp.int32(0), jnp.int32(CHUNK // L), vec_body, 0)


def _tile_body(tok_hbm, bank_hbm, out_hbm, tok_v, idx0, idx1, rows0, rows1,
               rows2, sem0, sem1, sem2):
    idx = (idx0, idx1)
    rows = (rows0, rows1, rows2)
    sems = (sem0, sem1, sem2)
    wid = (lax.axis_index("s").astype(jnp.int32) * jnp.int32(NC)
           + lax.axis_index("c").astype(jnp.int32))
    base = pl.multiple_of(wid * jnp.int32(PER_W), 128)
    for j in range(N_GRAM):
        pltpu.sync_copy(tok_hbm.at[jnp.int32(j), pl.ds(base, PER_W)],
                        tok_v.at[jnp.int32(j)])
    b_base = wid * jnp.int32(B_PER_W)

    def start_gather(r):
        seg = idx[(r // 2) % 2].at[pl.ds((r % 2) * S_DIM, S_DIM)]
        return pltpu.async_copy(bank_hbm.at[seg], rows[r % 3], sems[r % 3])

    _hash_chunk(tok_v, idx[0], 0)
    inflight = [start_gather(0), start_gather(1)]
    _hash_chunk(tok_v, idx[1], 1)
    for r in range(B_PER_W):
        if r + 2 < B_PER_W:
            if (r + 2) % 2 == 0 and r + 2 >= 4:
                _hash_chunk(tok_v, idx[((r + 2) // 2) % 2], (r + 2) // 2)
            inflight.append(start_gather(r + 2))
        inflight.pop(0).wait()
        pltpu.sync_copy(rows[r % 3], out_hbm.at[b_base + jnp.int32(r)])


_sc_gather = functools.partial(
    pl.kernel,
    mesh=plsc.VectorSubcoreMesh(core_axis_name="c", subcore_axis_name="s"),
    compiler_params=pltpu.CompilerParams(needs_layout_passes=False,
                                         use_tc_tiling_on_sc=True),
    out_type=jax.ShapeDtypeStruct((B_DIM, S_DIM, N_BLADES * D_STATE),
                                  jnp.float32),
    scratch_types=[
        pltpu.VMEM((N_GRAM, PER_W), jnp.int32),
        pltpu.VMEM((CHUNK,), jnp.int32),
        pltpu.VMEM((CHUNK,), jnp.int32),
        pltpu.VMEM((S_DIM, N_BLADES * D_STATE), jnp.float32),
        pltpu.VMEM((S_DIM, N_BLADES * D_STATE), jnp.float32),
        pltpu.VMEM((S_DIM, N_BLADES * D_STATE), jnp.float32),
        pltpu.SemaphoreType.DMA,
        pltpu.SemaphoreType.DMA,
        pltpu.SemaphoreType.DMA,
    ],
)(_tile_body)


def kernel(token_window, bank):
    tok = token_window.astype(jnp.int32).reshape(-1, N_GRAM).T
    bank_t = jnp.transpose(bank, (1, 0, 2)).reshape(N_SLOTS,
                                                    N_BLADES * D_STATE)
    out = _sc_gather(tok, bank_t)
    return out.reshape(B_DIM, S_DIM, N_BLADES, D_STATE)
